# baseline (speedup 1.0000x reference)
import jax
import jax.numpy as jnp
from jax.experimental import pallas as pl
from jax.experimental.pallas import tpu as pltpu

N_NODES = 50000
N_DAGS = 512
N_GLOBAL = 32
IN_DIM = 128
H = 64

BK = 5000
N_BLOCKS = N_NODES // BK
N_SUB = 5
SUB = BK // N_SUB


def _act(v):
    return jnp.maximum(v, 0.01 * v)


def _dot(a, b, dims):
    return jax.lax.dot_general(
        a, b, (dims, ((), ())),
        precision=jax.lax.Precision.DEFAULT,
        preferred_element_type=jnp.float32)


def _chunk_copy(smt_hbm, smt_vmem, sem, block, chunk, slot):
    rows = block * BK + chunk * SUB
    return pltpu.make_async_copy(
        smt_hbm.at[pl.ds(rows, SUB), :],
        smt_vmem.at[slot, chunk],
        sem.at[slot, chunk])


def _fused_kernel(x_ref, smt_hbm, rd_ref,
                  w1t_ref, b1_ref, w2_ref, b2_ref, w3_ref, b3_ref,
                  w4_ref, b4_ref, w5_ref, b5_ref, w6_ref, b6_ref,
                  s1_ref, s2_ref, smt_vmem, sem):
    k = pl.program_id(0)
    slot = jax.lax.rem(k, 2)

    @pl.when(k == 0)
    def _start_first():
        for c in range(N_SUB):
            _chunk_copy(smt_hbm, smt_vmem, sem, 0, c, 0).start()

    @pl.when(k + 1 < N_BLOCKS)
    def _prefetch_next():
        nslot = jax.lax.rem(k + 1, 2)
        for c in range(N_SUB):
            _chunk_copy(smt_hbm, smt_vmem, sem, k + 1, c, nslot).start()

    parts = []
    for c in range(N_SUB):
        x = x_ref[pl.ds(c * SUB, SUB), :]
        s = _act(_dot(x, w1t_ref[...], ((1,), (1,))) + b1_ref[...])
        s = _act(_dot(s, w2_ref[...], ((1,), (0,))) + b2_ref[...])
        s = _act(_dot(s, w3_ref[...], ((1,), (0,))) + b3_ref[...])
        _chunk_copy(smt_hbm, smt_vmem, sem, k, c, slot).wait()
        parts.append(_dot(s, smt_vmem[slot, c], ((0,), (0,))))
    part = sum(parts)

    @pl.when(k == 0)
    def _init():
        s1_ref[...] = part

    @pl.when(k != 0)
    def _acc():
        s1_ref[...] += part

    @pl.when(k == N_BLOCKS - 1)
    def _epilogue():
        s1t = s1_ref[...]
        g = _act(_dot(s1t, w4_ref[...], ((0,), (0,))) + b4_ref[...])
        g = _act(_dot(g, w5_ref[...], ((1,), (0,))) + b5_ref[...])
        g = _act(_dot(g, w6_ref[...], ((1,), (0,))) + b6_ref[...])
        s2_ref[...] = _dot(rd_ref[...], g, ((1,), (0,)))


@jax.jit
def kernel(summ_mats, running_dags_mat, inputs,
           W1, b1, W2, b2, W3, b3, W4, b4, W5, b5, W6, b6):
    full = lambda shape: pl.BlockSpec(shape, lambda k: (0, 0))
    biases = [b.reshape(1, H) for b in (b1, b2, b3, b4, b5, b6)]

    in_specs = [
        pl.BlockSpec((BK, IN_DIM), lambda k: (k, 0)),
        pl.BlockSpec(memory_space=pl.ANY),
        full((N_GLOBAL, N_DAGS)),
        full((H, IN_DIM)), full((1, H)),
        full((H, H)), full((1, H)),
        full((H, H)), full((1, H)),
        full((H, H)), full((1, H)),
        full((H, H)), full((1, H)),
        full((H, H)), full((1, H)),
    ]
    out_specs = [
        full((H, N_DAGS)),
        full((N_GLOBAL, H)),
    ]
    out_shapes = [
        jax.ShapeDtypeStruct((H, N_DAGS), jnp.float32),
        jax.ShapeDtypeStruct((N_GLOBAL, H), jnp.float32),
    ]

    s1t, s2 = pl.pallas_call(
        _fused_kernel,
        grid=(N_BLOCKS,),
        in_specs=in_specs,
        out_specs=out_specs,
        out_shape=out_shapes,
        scratch_shapes=[
            pltpu.VMEM((2, N_SUB, SUB, N_DAGS), jnp.float32),
            pltpu.SemaphoreType.DMA((2, N_SUB)),
        ],
    )(inputs, summ_mats.T, running_dags_mat,
      W1.T, biases[0], W2, biases[1], W3, biases[2],
      W4, biases[3], W5, biases[4], W6, biases[5])
    return (s1t.T, s2)


# speedup vs baseline: 1.1119x; 1.1119x over previous
import jax
import jax.numpy as jnp
from jax.experimental import pallas as pl

N_NODES = 50000
N_DAGS = 512
N_GLOBAL = 32
IN_DIM = 128
H = 64

BK = 5000
N_BLOCKS = N_NODES // BK
N_SUB = 5
SUB = BK // N_SUB


def _act(v):
    return jnp.maximum(v, 0.01 * v)


def _dot(a, b, dims):
    return jax.lax.dot_general(
        a, b, (dims, ((), ())),
        precision=jax.lax.Precision.DEFAULT,
        preferred_element_type=jnp.float32)


def _fused_kernel(x_ref, smt_ref, rd_ref,
                  w1t_ref, b1_ref, w2_ref, b2_ref, w3_ref, b3_ref,
                  w4_ref, b4_ref, w5_ref, b5_ref, w6_ref, b6_ref,
                  s1_ref, s2_ref):
    k = pl.program_id(0)

    parts = []
    for c in range(N_SUB):
        x = x_ref[pl.ds(c * SUB, SUB), :]
        s = _act(_dot(x, w1t_ref[...], ((1,), (1,))) + b1_ref[...])
        s = _act(_dot(s, w2_ref[...], ((1,), (0,))) + b2_ref[...])
        s = _act(_dot(s, w3_ref[...], ((1,), (0,))) + b3_ref[...])
        parts.append(_dot(s, smt_ref[pl.ds(c * SUB, SUB), :], ((0,), (0,))))
    part = sum(parts)

    @pl.when(k == 0)
    def _init():
        s1_ref[...] = part

    @pl.when(k != 0)
    def _acc():
        s1_ref[...] += part

    @pl.when(k == N_BLOCKS - 1)
    def _epilogue():
        s1t = s1_ref[...]
        g = _act(_dot(s1t, w4_ref[...], ((0,), (0,))) + b4_ref[...])
        g = _act(_dot(g, w5_ref[...], ((1,), (0,))) + b5_ref[...])
        g = _act(_dot(g, w6_ref[...], ((1,), (0,))) + b6_ref[...])
        s2_ref[...] = _dot(rd_ref[...], g, ((1,), (0,)))


@jax.jit
def kernel(summ_mats, running_dags_mat, inputs,
           W1, b1, W2, b2, W3, b3, W4, b4, W5, b5, W6, b6):
    full = lambda shape: pl.BlockSpec(shape, lambda k: (0, 0))
    biases = [b.reshape(1, H) for b in (b1, b2, b3, b4, b5, b6)]

    in_specs = [
        pl.BlockSpec((BK, IN_DIM), lambda k: (k, 0)),
        pl.BlockSpec((BK, N_DAGS), lambda k: (k, 0)),
        full((N_GLOBAL, N_DAGS)),
        full((H, IN_DIM)), full((1, H)),
        full((H, H)), full((1, H)),
        full((H, H)), full((1, H)),
        full((H, H)), full((1, H)),
        full((H, H)), full((1, H)),
        full((H, H)), full((1, H)),
    ]
    out_specs = [
        full((H, N_DAGS)),
        full((N_GLOBAL, H)),
    ]
    out_shapes = [
        jax.ShapeDtypeStruct((H, N_DAGS), jnp.float32),
        jax.ShapeDtypeStruct((N_GLOBAL, H), jnp.float32),
    ]

    s1t, s2 = pl.pallas_call(
        _fused_kernel,
        grid=(N_BLOCKS,),
        in_specs=in_specs,
        out_specs=out_specs,
        out_shape=out_shapes,
    )(inputs, summ_mats.T, running_dags_mat,
      W1.T, biases[0], W2, biases[1], W3, biases[2],
      W4, biases[3], W5, biases[4], W6, biases[5])
    return (s1t.T, s2)
